# baseline (speedup 1.0000x reference)
import functools

import jax
import jax.numpy as jnp
from jax import lax
from jax.experimental import pallas as pl
from jax.experimental.pallas import tpu as pltpu
from jax.experimental.pallas import tpu_sc as plsc

N = 4096
K = 8192
BN = 1024
NB = N // BN
BK = 2048
KC = K // BK

NC = 2
NS = 16
NW = NC * NS
QPW = N // NW
L = 16


def _argmin_body(qt_ref, r_ref, idx_ref):
    qx = qt_ref[0:1, :]
    qy = qt_ref[1:2, :]
    m_run = None
    fid_run = None
    for c in range(KC):
        rx = r_ref[pl.ds(c * BK, BK), 0:1]
        ry = r_ref[pl.ds(c * BK, BK), 1:2]
        dx = rx - qx
        dy = ry - qy
        d2 = dx * dx + dy * dy
        m_c = jnp.min(d2, axis=0, keepdims=True)
        fiota = jax.lax.broadcasted_iota(
            jnp.int32, (BK, BN), 0).astype(jnp.float32)
        fid_c = jnp.min(jnp.where(d2 <= m_c, fiota, float(BK)), axis=0,
                        keepdims=True) + float(c * BK)
        if m_run is None:
            m_run, fid_run = m_c, fid_c
        else:
            upd = m_c < m_run
            fid_run = jnp.where(upd, fid_c, fid_run)
            m_run = jnp.where(upd, m_c, m_run)
    idx_ref[...] = fid_run.astype(jnp.int32).reshape(BN // 128, 128)


def _sc_gather_body(idx_hbm, dp_hbm, b_hbm, nw_hbm,
                    d2_out, b_out,
                    idx_v, dp_v, b_v, nw_v, d2_v, gb_v):
    wid = lax.axis_index("s") * NC + lax.axis_index("c")
    base = wid * QPW
    pltpu.sync_copy(idx_hbm.at[wid], idx_v)
    pltpu.sync_copy(dp_hbm, dp_v)
    pltpu.sync_copy(b_hbm, b_v)
    pltpu.sync_copy(nw_hbm.at[pl.ds(2 * base, 2 * QPW)], nw_v)
    li = lax.iota(jnp.int32, L)
    for j in range(QPW // L):
        sl = pl.ds(j * L, L)
        iv = idx_v[sl]
        iv2 = iv * 2
        gx = plsc.load_gather(dp_v, [iv2])
        gy = plsc.load_gather(dp_v, [iv2 + 1])
        gb = plsc.load_gather(b_v, [iv])
        nl = li * 2 + (2 * L) * j
        nx = plsc.load_gather(nw_v, [nl])
        ny = plsc.load_gather(nw_v, [nl + 1])
        dx = nx - gx
        dy = ny - gy
        d2_v[sl] = dx * dx + dy * dy
        gb_v[sl] = gb
    pltpu.sync_copy(d2_v, d2_out.at[wid])
    pltpu.sync_copy(gb_v, b_out.at[wid])


def _reduce_body(d2_ref, b_ref, out_ref):
    pp = jnp.sqrt(d2_ref[...])
    b = b_ref[...]
    out_ref[0, 0] = jnp.sum(pp * b) / jnp.sum(b)


@jax.jit
def _track_loss(qt, dict_ref, dpflat, bf, nwflat):
    idx = pl.pallas_call(
        _argmin_body,
        grid=(NB,),
        in_specs=[
            pl.BlockSpec((2, BN), lambda nb: (0, nb)),
            pl.BlockSpec((K, 2), lambda nb: (0, 0)),
        ],
        out_specs=pl.BlockSpec((BN // 128, 128), lambda nb: (nb, 0)),
        out_shape=jax.ShapeDtypeStruct((NW, QPW), jnp.int32),
        compiler_params=pltpu.CompilerParams(
            dimension_semantics=("arbitrary",),
        ),
    )(qt, dict_ref)

    sc_gather = functools.partial(
        pl.kernel,
        out_type=(
            jax.ShapeDtypeStruct((NW, QPW), jnp.float32),
            jax.ShapeDtypeStruct((NW, QPW), jnp.float32),
        ),
        mesh=plsc.VectorSubcoreMesh(core_axis_name="c", subcore_axis_name="s"),
        compiler_params=pltpu.CompilerParams(needs_layout_passes=False),
        scratch_types=[
            pltpu.VMEM((QPW,), jnp.int32),
            pltpu.VMEM((2 * K,), jnp.float32),
            pltpu.VMEM((K,), jnp.float32),
            pltpu.VMEM((2 * QPW,), jnp.float32),
            pltpu.VMEM((QPW,), jnp.float32),
            pltpu.VMEM((QPW,), jnp.float32),
        ],
    )(_sc_gather_body)
    d2g, bg = sc_gather(idx, dpflat, bf, nwflat)

    out = pl.pallas_call(
        _reduce_body,
        out_specs=pl.BlockSpec(memory_space=pltpu.SMEM),
        out_shape=jax.ShapeDtypeStruct((1, 1), jnp.float32),
    )(d2g, bg)
    return out[0, 0]


def kernel(flat_origin_curves, flat_new_curves, dict_points, dict_ref, dict_bool):
    qt = flat_origin_curves.T
    bf = dict_bool.astype(jnp.float32)
    dpflat = dict_points.reshape(2 * K)
    nwflat = flat_new_curves.reshape(2 * N)
    return _track_loss(qt, dict_ref, dpflat, bf, nwflat)


# speedup vs baseline: 1.0290x; 1.0290x over previous
import functools

import jax
import jax.numpy as jnp
from jax import lax
from jax.experimental import pallas as pl
from jax.experimental.pallas import tpu as pltpu
from jax.experimental.pallas import tpu_sc as plsc

N = 4096
K = 8192
BN = 1024
NB = N // BN
BK = 2048
KC = K // BK

NC = 2
NS = 16
NW = NC * NS
QPW = N // NW
L = 16


def _argmin_body(qt_ref, r_ref, idx_ref):
    qx = qt_ref[0:1, :]
    qy = qt_ref[1:2, :]
    m_run = None
    fid_run = None
    for c in range(KC):
        rx = r_ref[pl.ds(c * BK, BK), 0:1]
        ry = r_ref[pl.ds(c * BK, BK), 1:2]
        dx = rx - qx
        dy = ry - qy
        d2 = dx * dx + dy * dy
        m_c = jnp.min(d2, axis=0, keepdims=True)
        fiota = jax.lax.broadcasted_iota(
            jnp.int32, (BK, BN), 0).astype(jnp.float32)
        fid_c = jnp.min(jnp.where(d2 <= m_c, fiota, float(BK)), axis=0,
                        keepdims=True) + float(c * BK)
        if m_run is None:
            m_run, fid_run = m_c, fid_c
        else:
            upd = m_c < m_run
            fid_run = jnp.where(upd, fid_c, fid_run)
            m_run = jnp.where(upd, m_c, m_run)
    idx_ref[...] = fid_run.astype(jnp.int32).reshape(BN // 128, 128)


def _sc_gather_body(idx_hbm, dall_hbm,
                    d2_out, b_out,
                    idx_v, dp_v, b_v, nw_v, d2_v, gb_v, s0, s1, s2, s3):
    wid = lax.axis_index("s") * NC + lax.axis_index("c")
    base = wid * QPW
    c0 = pltpu.async_copy(idx_hbm.at[wid], idx_v, s0)
    c1 = pltpu.async_copy(dall_hbm.at[pl.ds(0, 2 * K)], dp_v, s1)
    c2 = pltpu.async_copy(dall_hbm.at[pl.ds(2 * K, K)], b_v, s2)
    c3 = pltpu.async_copy(
        dall_hbm.at[pl.ds(3 * K + 2 * base, 2 * QPW)], nw_v, s3)
    c0.wait()
    c1.wait()
    c2.wait()
    c3.wait()
    li = lax.iota(jnp.int32, L)
    for j in range(QPW // L):
        sl = pl.ds(j * L, L)
        iv = idx_v[sl]
        iv2 = iv * 2
        gx = plsc.load_gather(dp_v, [iv2])
        gy = plsc.load_gather(dp_v, [iv2 + 1])
        gb = plsc.load_gather(b_v, [iv])
        nl = li * 2 + (2 * L) * j
        nx = plsc.load_gather(nw_v, [nl])
        ny = plsc.load_gather(nw_v, [nl + 1])
        dx = nx - gx
        dy = ny - gy
        d2_v[sl] = dx * dx + dy * dy
        gb_v[sl] = gb
    pltpu.sync_copy(d2_v, d2_out.at[wid])
    pltpu.sync_copy(gb_v, b_out.at[wid])


def _reduce_body(d2_ref, b_ref, out_ref):
    pp = jnp.sqrt(d2_ref[...])
    b = b_ref[...]
    out_ref[0, 0] = jnp.sum(pp * b) / jnp.sum(b)


@jax.jit
def _track_loss(qt, dict_ref, dall):
    idx = pl.pallas_call(
        _argmin_body,
        grid=(NB,),
        in_specs=[
            pl.BlockSpec((2, BN), lambda nb: (0, nb)),
            pl.BlockSpec((K, 2), lambda nb: (0, 0)),
        ],
        out_specs=pl.BlockSpec((BN // 128, 128), lambda nb: (nb, 0)),
        out_shape=jax.ShapeDtypeStruct((NW, QPW), jnp.int32),
        compiler_params=pltpu.CompilerParams(
            dimension_semantics=("arbitrary",),
        ),
    )(qt, dict_ref)

    sc_gather = functools.partial(
        pl.kernel,
        out_type=(
            jax.ShapeDtypeStruct((NW, QPW), jnp.float32),
            jax.ShapeDtypeStruct((NW, QPW), jnp.float32),
        ),
        mesh=plsc.VectorSubcoreMesh(core_axis_name="c", subcore_axis_name="s"),
        compiler_params=pltpu.CompilerParams(needs_layout_passes=False),
        scratch_types=[
            pltpu.VMEM((QPW,), jnp.int32),
            pltpu.VMEM((2 * K,), jnp.float32),
            pltpu.VMEM((K,), jnp.float32),
            pltpu.VMEM((2 * QPW,), jnp.float32),
            pltpu.VMEM((QPW,), jnp.float32),
            pltpu.VMEM((QPW,), jnp.float32),
            pltpu.SemaphoreType.DMA,
            pltpu.SemaphoreType.DMA,
            pltpu.SemaphoreType.DMA,
            pltpu.SemaphoreType.DMA,
        ],
    )(_sc_gather_body)
    d2g, bg = sc_gather(idx, dall)

    out = pl.pallas_call(
        _reduce_body,
        out_specs=pl.BlockSpec(memory_space=pltpu.SMEM),
        out_shape=jax.ShapeDtypeStruct((1, 1), jnp.float32),
    )(d2g, bg)
    return out[0, 0]


def kernel(flat_origin_curves, flat_new_curves, dict_points, dict_ref, dict_bool):
    qt = flat_origin_curves.T
    dall = jnp.concatenate([
        dict_points.reshape(2 * K),
        dict_bool.astype(jnp.float32),
        flat_new_curves.reshape(2 * N),
    ])
    return _track_loss(qt, dict_ref, dall)
